# baseline (speedup 1.0000x reference)
import jax
import jax.numpy as jnp
from jax import lax
from jax.experimental import pallas as pl
from jax.experimental.pallas import tpu as pltpu
from jax.experimental.pallas import tpu_sc as plsc

B, S, CIN = 4, 8192, 1024
STRIDE = 4
COUT = CIN // STRIDE
ROWS = B * S

SC_ROWS = ROWS // 2
TC_ROWS = ROWS - SC_ROWS

NC, NS = 2, 16
NW = NC * NS
RW = SC_ROWS // NW

R = 32
NCH = RW // R


def _sc_body(x_hbm, out_hbm, in0, in1, ob0, ob1, si0, si1, so0, so1):
    wid = lax.axis_index("s") * NC + lax.axis_index("c")
    row0 = wid * RW
    lanes4 = lax.iota(jnp.int32, 16) * STRIDE
    ins, obs, sis, sos = (in0, in1), (ob0, ob1), (si0, si1), (so0, so1)

    def in_slice(c):
        return x_hbm.at[pl.ds(row0 + c * R, R), :]

    def out_slice(c):
        return out_hbm.at[pl.ds(row0 + c * R, R), :]

    pltpu.async_copy(in_slice(0), ins[0], sis[0])

    @pl.loop(0, NCH // 2)
    def _pair(p):
        for b in range(2):
            c = p * 2 + b

            @pl.when(c + 1 < NCH)
            def _start_next_in():
                pltpu.async_copy(in_slice(c + 1), ins[1 - b], sis[1 - b])

            pltpu.make_async_copy(in_slice(c), ins[b], sis[b]).wait()

            @pl.loop(0, R, unroll=4)
            def _row(r):
                ridx = jnp.full((16,), r, jnp.int32)
                for j in range(COUT // 16):
                    v = plsc.load_gather(ins[b], [ridx, lanes4 + j * (16 * STRIDE)])
                    obs[b][r, pl.ds(j * 16, 16)] = v

            @pl.when(c >= 2)
            def _drain_prev_out():
                pltpu.make_async_copy(obs[b], out_slice(c - 2), sos[b]).wait()

            pltpu.async_copy(obs[b], out_slice(c), sos[b])

    for b in range(2):
        pltpu.make_async_copy(obs[b], out_slice(NCH - 2 + b), sos[b]).wait()


_mesh = plsc.VectorSubcoreMesh(core_axis_name="c", subcore_axis_name="s")

_sc_sel = pl.kernel(
    _sc_body,
    out_type=jax.ShapeDtypeStruct((SC_ROWS, COUT), jnp.float32),
    mesh=_mesh,
    scratch_types=[
        pltpu.VMEM((R, CIN), jnp.float32),
        pltpu.VMEM((R, CIN), jnp.float32),
        pltpu.VMEM((R, COUT), jnp.float32),
        pltpu.VMEM((R, COUT), jnp.float32),
        pltpu.SemaphoreType.DMA,
        pltpu.SemaphoreType.DMA,
        pltpu.SemaphoreType.DMA,
        pltpu.SemaphoreType.DMA,
    ],
    compiler_params=pltpu.CompilerParams(
        use_tc_tiling_on_sc=True,
        needs_layout_passes=False,
    ),
)

RB = 512
GRID = TC_ROWS // RB


def _tc_body(x_ref, o_ref):
    lane = jax.lax.broadcasted_iota(jnp.int32, (RB, 128), 1)
    idx = (lane % 32) * STRIDE
    for g in range(CIN // 128):
        src = x_ref[:, g * 128:(g + 1) * 128]
        gathered = jnp.take_along_axis(src, idx, axis=1)
        o_ref[:, g * 32:(g + 1) * 32] = gathered[:, :32]


_tc_sel = pl.pallas_call(
    _tc_body,
    grid=(GRID,),
    in_specs=[pl.BlockSpec((RB, CIN), lambda i: (i, 0))],
    out_specs=pl.BlockSpec((RB, COUT), lambda i: (i, 0)),
    out_shape=jax.ShapeDtypeStruct((TC_ROWS, COUT), jnp.float32),
)


def kernel(x):
    x2 = x.reshape(ROWS, CIN)
    sc_out = _sc_sel(x2[:SC_ROWS])
    tc_out = _tc_sel(x2[SC_ROWS:])
    return jnp.concatenate([sc_out, tc_out], axis=0).reshape(B, S, COUT)


# speedup vs baseline: 1.9790x; 1.9790x over previous
import jax
import jax.numpy as jnp
from jax import lax
from jax.experimental import pallas as pl
from jax.experimental.pallas import tpu as pltpu
from jax.experimental.pallas import tpu_sc as plsc

B, S, CIN = 4, 8192, 1024
STRIDE = 4
COUT = CIN // STRIDE
ROWS = B * S

SC_ROWS = ROWS // 2
TC_ROWS = ROWS - SC_ROWS

NC, NS = 2, 16
NW = NC * NS
RW = SC_ROWS // NW

R = 32
NCH = RW // R


def _sc_body(x_hbm, out_hbm, in0, in1, ob0, ob1, si0, si1, so0, so1):
    wid = lax.axis_index("s") * NC + lax.axis_index("c")
    row0 = wid * RW
    lanes4 = lax.iota(jnp.int32, 16) * STRIDE
    ins, obs, sis, sos = (in0, in1), (ob0, ob1), (si0, si1), (so0, so1)

    def in_slice(c):
        return x_hbm.at[pl.ds(row0 + c * R, R), :]

    def out_slice(c):
        return out_hbm.at[pl.ds(row0 + c * R, R), :]

    pltpu.async_copy(in_slice(0), ins[0], sis[0])

    @pl.loop(0, NCH // 2)
    def _pair(p):
        for b in range(2):
            c = p * 2 + b

            @pl.when(c + 1 < NCH)
            def _start_next_in():
                pltpu.async_copy(in_slice(c + 1), ins[1 - b], sis[1 - b])

            pltpu.make_async_copy(in_slice(c), ins[b], sis[b]).wait()

            @pl.loop(0, R, unroll=4)
            def _row(r):
                ridx = jnp.full((16,), r, jnp.int32)
                for j in range(COUT // 16):
                    v = plsc.load_gather(ins[b], [ridx, lanes4 + j * (16 * STRIDE)])
                    obs[b][r, pl.ds(j * 16, 16)] = v

            @pl.when(c >= 2)
            def _drain_prev_out():
                pltpu.make_async_copy(obs[b], out_slice(c - 2), sos[b]).wait()

            pltpu.async_copy(obs[b], out_slice(c), sos[b])

    for b in range(2):
        pltpu.make_async_copy(obs[b], out_slice(NCH - 2 + b), sos[b]).wait()


_mesh = plsc.VectorSubcoreMesh(core_axis_name="c", subcore_axis_name="s")

_sc_sel = pl.kernel(
    _sc_body,
    out_type=jax.ShapeDtypeStruct((SC_ROWS, COUT), jnp.float32),
    mesh=_mesh,
    scratch_types=[
        pltpu.VMEM((R, CIN), jnp.float32),
        pltpu.VMEM((R, CIN), jnp.float32),
        pltpu.VMEM((R, COUT), jnp.float32),
        pltpu.VMEM((R, COUT), jnp.float32),
        pltpu.SemaphoreType.DMA,
        pltpu.SemaphoreType.DMA,
        pltpu.SemaphoreType.DMA,
        pltpu.SemaphoreType.DMA,
    ],
    compiler_params=pltpu.CompilerParams(
        use_tc_tiling_on_sc=True,
        needs_layout_passes=False,
    ),
)

RB = 512
GRID = TC_ROWS // RB


def _tc_body(x_ref, o_ref):
    lane = jax.lax.broadcasted_iota(jnp.int32, (RB, 128), 1)
    idx = (lane % 32) * STRIDE
    for g in range(CIN // 128):
        src = x_ref[:, g * 128:(g + 1) * 128]
        gathered = jnp.take_along_axis(src, idx, axis=1)
        o_ref[:, g * 32:(g + 1) * 32] = gathered[:, :32]


_TC_OFF = SC_ROWS // RB

_tc_sel = pl.pallas_call(
    _tc_body,
    grid=(GRID,),
    in_specs=[pl.BlockSpec((RB, CIN), lambda i: (i + _TC_OFF, 0))],
    out_specs=pl.BlockSpec((RB, COUT), lambda i: (i, 0)),
    out_shape=jax.ShapeDtypeStruct((TC_ROWS, COUT), jnp.float32),
)


def kernel(x):
    x2 = x.reshape(ROWS, CIN)
    sc_out = _sc_sel(x2)
    tc_out = _tc_sel(x2)
    return jnp.concatenate([sc_out, tc_out], axis=0).reshape(B, S, COUT)


# speedup vs baseline: 2.1021x; 1.0622x over previous
import jax
import jax.numpy as jnp
from jax import lax
from jax.experimental import pallas as pl
from jax.experimental.pallas import tpu as pltpu
from jax.experimental.pallas import tpu_sc as plsc

B, S, CIN = 4, 8192, 1024
STRIDE = 4
COUT = CIN // STRIDE
ROWS = B * S

NC, NS = 2, 16
NW = NC * NS
RW = ROWS // NW

R = 32
NCH = RW // R


def _selector_body(x_hbm, out_hbm, in0, in1, ob0, ob1, si0, si1, so0, so1):
    wid = lax.axis_index("s") * NC + lax.axis_index("c")
    row0 = wid * RW
    lanes4 = lax.iota(jnp.int32, 16) * STRIDE
    ins, obs, sis, sos = (in0, in1), (ob0, ob1), (si0, si1), (so0, so1)

    def in_slice(c):
        return x_hbm.at[pl.ds(row0 + c * R, R), :]

    def out_slice(c):
        return out_hbm.at[pl.ds(row0 + c * R, R), :]

    pltpu.async_copy(in_slice(0), ins[0], sis[0])

    @pl.loop(0, NCH // 2)
    def _pair(p):
        for b in range(2):
            c = p * 2 + b

            @pl.when(c + 1 < NCH)
            def _start_next_in():
                pltpu.async_copy(in_slice(c + 1), ins[1 - b], sis[1 - b])

            with jax.named_scope("wait_in"):
                pltpu.make_async_copy(in_slice(c), ins[b], sis[b]).wait()

            with jax.named_scope("gather"):
                @pl.loop(0, R, unroll=4)
                def _row(r):
                    ridx = jnp.full((16,), r, jnp.int32)
                    for j in range(COUT // 16):
                        v = plsc.load_gather(ins[b], [ridx, lanes4 + j * (16 * STRIDE)])
                        obs[b][r, pl.ds(j * 16, 16)] = v

            with jax.named_scope("wait_out"):
                @pl.when(c >= 2)
                def _drain_prev_out():
                    pltpu.make_async_copy(obs[b], out_slice(c - 2), sos[b]).wait()

            pltpu.async_copy(obs[b], out_slice(c), sos[b])

    for b in range(2):
        pltpu.make_async_copy(obs[b], out_slice(NCH - 2 + b), sos[b]).wait()


_mesh = plsc.VectorSubcoreMesh(core_axis_name="c", subcore_axis_name="s")

_selector = pl.kernel(
    _selector_body,
    out_type=jax.ShapeDtypeStruct((ROWS, COUT), jnp.float32),
    mesh=_mesh,
    scratch_types=[
        pltpu.VMEM((R, CIN), jnp.float32),
        pltpu.VMEM((R, CIN), jnp.float32),
        pltpu.VMEM((R, COUT), jnp.float32),
        pltpu.VMEM((R, COUT), jnp.float32),
        pltpu.SemaphoreType.DMA,
        pltpu.SemaphoreType.DMA,
        pltpu.SemaphoreType.DMA,
        pltpu.SemaphoreType.DMA,
    ],
    compiler_params=pltpu.CompilerParams(
        use_tc_tiling_on_sc=True,
        needs_layout_passes=False,
    ),
)


def kernel(x):
    out2 = _selector(x.reshape(ROWS, CIN))
    return out2.reshape(B, S, COUT)
